# Initial kernel scaffold; baseline (speedup 1.0000x reference)
import jax
import jax.numpy as jnp
from jax.experimental import pallas as pl


def kernel(x, x_importance, edge_index, bn_gamma, bn_beta, W1, b1, W2, b2, W3, b3, W4, b4, W5, b5, fc_w):
    raise NotImplementedError("write your pallas kernel here")


# speedup vs baseline: 4.4316x; 4.4316x over previous
import functools

import jax
import jax.numpy as jnp
from jax import lax
from jax.experimental import pallas as pl
from jax.experimental.pallas import tpu as pltpu
from jax.experimental.pallas import tpu_sc as plsc

N = 10000
E = 320000
D = 128

NC = 2
NS = 16
NW = NC * NS
EPW = E // NW
K = 80
NCHUNK = EPW // K


def _agg_body(h_hbm, src_hbm, dst_hbm, zero_hbm, out_hbm,
              idx_s, idx_d, rows, acc, sem):
    cid = lax.axis_index("c")
    sid = lax.axis_index("s")
    wid = sid * NC + cid

    @pl.when(sid == 0)
    def _():
        pltpu.sync_copy(zero_hbm, acc)

    plsc.subcore_barrier()

    def body(c, carry):
        base = wid * EPW + c * K
        pltpu.sync_copy(src_hbm.at[pl.ds(base, K)], idx_s)
        pltpu.sync_copy(dst_hbm.at[pl.ds(base, K)], idx_d)
        pltpu.async_copy(h_hbm.at[idx_s], rows, sem).wait()
        pltpu.sync_copy(rows, acc.at[idx_d], add=True)
        return carry

    lax.fori_loop(0, NCHUNK, body, 0)

    plsc.subcore_barrier()

    @pl.when(sid == 0)
    def _():
        pltpu.sync_copy(acc, out_hbm.at[cid])


_agg_call = pl.kernel(
    _agg_body,
    out_type=jax.ShapeDtypeStruct((NC, N, D), jnp.float32),
    mesh=plsc.VectorSubcoreMesh(core_axis_name="c", subcore_axis_name="s"),
    scratch_types=[
        pltpu.VMEM((K,), jnp.int32),
        pltpu.VMEM((K,), jnp.int32),
        pltpu.VMEM((K, D), jnp.float32),
        pltpu.VMEM_SHARED((N, D), jnp.float32),
        pltpu.SemaphoreType.DMA,
    ],
)


def _bn_kernel(x_ref, imp_ref, g_ref, b_ref, o_ref):
    xs = x_ref[...] * imp_ref[...]
    mean = jnp.mean(xs, axis=0, keepdims=True)
    cent = xs - mean
    var = jnp.mean(cent * cent, axis=0, keepdims=True)
    o_ref[...] = cent * lax.rsqrt(var + 1e-5) * g_ref[...] + b_ref[...]


_bn_call = pl.pallas_call(
    _bn_kernel,
    out_shape=jax.ShapeDtypeStruct((N, D), jnp.float32),
)


BN_ROWS = 2000


def _layer_kernel(h_ref, a_ref, w_ref, b_ref, o_ref):
    s = h_ref[...] + a_ref[0] + a_ref[1]
    o_ref[...] = jnp.tanh(
        jnp.dot(s, w_ref[...], preferred_element_type=jnp.float32) + b_ref[...])


_layer_call = pl.pallas_call(
    _layer_kernel,
    grid=(N // BN_ROWS,),
    in_specs=[
        pl.BlockSpec((BN_ROWS, D), lambda i: (i, 0)),
        pl.BlockSpec((NC, BN_ROWS, D), lambda i: (0, i, 0)),
        pl.BlockSpec((D, D), lambda i: (0, 0)),
        pl.BlockSpec((1, D), lambda i: (0, 0)),
    ],
    out_specs=pl.BlockSpec((BN_ROWS, D), lambda i: (i, 0)),
    out_shape=jax.ShapeDtypeStruct((N, D), jnp.float32),
)


def _last_kernel(h_ref, a_ref, w_ref, b_ref, fc_ref, h5_ref, h6_ref):
    s = h_ref[...] + a_ref[0] + a_ref[1]
    h5 = jnp.tanh(
        jnp.dot(s, w_ref[...], preferred_element_type=jnp.float32) + b_ref[...])
    h5_ref[...] = h5
    h6_ref[...] = jnp.tanh(
        jnp.dot(h5, fc_ref[...], preferred_element_type=jnp.float32))


_last_call = pl.pallas_call(
    _last_kernel,
    grid=(N // BN_ROWS,),
    in_specs=[
        pl.BlockSpec((BN_ROWS, D), lambda i: (i, 0)),
        pl.BlockSpec((NC, BN_ROWS, D), lambda i: (0, i, 0)),
        pl.BlockSpec((D, D), lambda i: (0, 0)),
        pl.BlockSpec((1, D), lambda i: (0, 0)),
        pl.BlockSpec((D, D), lambda i: (0, 0)),
    ],
    out_specs=[
        pl.BlockSpec((BN_ROWS, D), lambda i: (i, 0)),
        pl.BlockSpec((BN_ROWS, D), lambda i: (i, 0)),
    ],
    out_shape=[
        jax.ShapeDtypeStruct((N, D), jnp.float32),
        jax.ShapeDtypeStruct((N, D), jnp.float32),
    ],
)


def kernel(x, x_importance, edge_index, bn_gamma, bn_beta,
           W1, b1, W2, b2, W3, b3, W4, b4, W5, b5, fc_w):
    src = edge_index[0]
    dst = edge_index[1]
    zeros = jnp.zeros((N, D), jnp.float32)

    h = _bn_call(x, x_importance, bn_gamma.reshape(1, D), bn_beta.reshape(1, D))

    hs = []
    for W, b in ((W1, b1), (W2, b2), (W3, b3), (W4, b4)):
        agg = _agg_call(h, src, dst, zeros)
        h = _layer_call(h, agg, W, b.reshape(1, D))
        hs.append(h)

    agg = _agg_call(h, src, dst, zeros)
    h5, h6 = _last_call(h, agg, W5, b5.reshape(1, D), fc_w)

    return jnp.concatenate([hs[0], hs[1], hs[2], hs[3], h5, h6], axis=-1)


# speedup vs baseline: 6.3930x; 1.4426x over previous
import functools

import jax
import jax.numpy as jnp
from jax import lax
from jax.experimental import pallas as pl
from jax.experimental.pallas import tpu as pltpu
from jax.experimental.pallas import tpu_sc as plsc

N = 10000
E = 320000
D = 128

NC = 2
NS = 16
NW = NC * NS
EPW = E // NW
K = 80
NCHUNK = EPW // K


def _agg_body(h_hbm, src_hbm, dst_hbm, zero_hbm, out_hbm,
              src_v, dst_v, rows, acc, sem):
    cid = lax.axis_index("c")
    sid = lax.axis_index("s")
    wid = sid * NC + cid

    pltpu.sync_copy(src_hbm.at[pl.ds(wid * EPW, EPW)], src_v)
    pltpu.sync_copy(dst_hbm.at[wid], dst_v)

    @pl.when(sid == 0)
    def _():
        pltpu.sync_copy(zero_hbm, acc)

    plsc.subcore_barrier()

    def body(c, carry):
        pltpu.async_copy(h_hbm.at[src_v.at[pl.ds(c * K, K)]], rows, sem).wait()
        pltpu.sync_copy(rows, acc.at[dst_v.at[c]], add=True)
        return carry

    lax.fori_loop(0, NCHUNK, body, 0)

    plsc.subcore_barrier()

    @pl.when(sid == 0)
    def _():
        pltpu.sync_copy(acc, out_hbm.at[cid])


_agg_call = pl.kernel(
    _agg_body,
    out_type=jax.ShapeDtypeStruct((NC, N, D), jnp.float32),
    mesh=plsc.VectorSubcoreMesh(core_axis_name="c", subcore_axis_name="s"),
    scratch_types=[
        pltpu.VMEM((EPW,), jnp.int32),
        pltpu.VMEM((NCHUNK, K), jnp.int32),
        pltpu.VMEM((K, D), jnp.float32),
        pltpu.VMEM_SHARED((N, D), jnp.float32),
        pltpu.SemaphoreType.DMA,
    ],
)


def _bn_kernel(x_ref, imp_ref, g_ref, b_ref, o_ref):
    xs = x_ref[...] * imp_ref[...]
    mean = jnp.mean(xs, axis=0, keepdims=True)
    cent = xs - mean
    var = jnp.mean(cent * cent, axis=0, keepdims=True)
    o_ref[...] = cent * lax.rsqrt(var + 1e-5) * g_ref[...] + b_ref[...]


_bn_call = pl.pallas_call(
    _bn_kernel,
    out_shape=jax.ShapeDtypeStruct((N, D), jnp.float32),
)


BN_ROWS = 2000


def _layer_kernel(h_ref, a_ref, w_ref, b_ref, o_ref):
    s = h_ref[...] + a_ref[0] + a_ref[1]
    o_ref[...] = jnp.tanh(
        jnp.dot(s, w_ref[...], preferred_element_type=jnp.float32) + b_ref[...])


_layer_call = pl.pallas_call(
    _layer_kernel,
    grid=(N // BN_ROWS,),
    in_specs=[
        pl.BlockSpec((BN_ROWS, D), lambda i: (i, 0)),
        pl.BlockSpec((NC, BN_ROWS, D), lambda i: (0, i, 0)),
        pl.BlockSpec((D, D), lambda i: (0, 0)),
        pl.BlockSpec((1, D), lambda i: (0, 0)),
    ],
    out_specs=pl.BlockSpec((BN_ROWS, D), lambda i: (i, 0)),
    out_shape=jax.ShapeDtypeStruct((N, D), jnp.float32),
)


def _last_kernel(h_ref, a_ref, w_ref, b_ref, fc_ref, h5_ref, h6_ref):
    s = h_ref[...] + a_ref[0] + a_ref[1]
    h5 = jnp.tanh(
        jnp.dot(s, w_ref[...], preferred_element_type=jnp.float32) + b_ref[...])
    h5_ref[...] = h5
    h6_ref[...] = jnp.tanh(
        jnp.dot(h5, fc_ref[...], preferred_element_type=jnp.float32))


_last_call = pl.pallas_call(
    _last_kernel,
    grid=(N // BN_ROWS,),
    in_specs=[
        pl.BlockSpec((BN_ROWS, D), lambda i: (i, 0)),
        pl.BlockSpec((NC, BN_ROWS, D), lambda i: (0, i, 0)),
        pl.BlockSpec((D, D), lambda i: (0, 0)),
        pl.BlockSpec((1, D), lambda i: (0, 0)),
        pl.BlockSpec((D, D), lambda i: (0, 0)),
    ],
    out_specs=[
        pl.BlockSpec((BN_ROWS, D), lambda i: (i, 0)),
        pl.BlockSpec((BN_ROWS, D), lambda i: (i, 0)),
    ],
    out_shape=[
        jax.ShapeDtypeStruct((N, D), jnp.float32),
        jax.ShapeDtypeStruct((N, D), jnp.float32),
    ],
)


def kernel(x, x_importance, edge_index, bn_gamma, bn_beta,
           W1, b1, W2, b2, W3, b3, W4, b4, W5, b5, fc_w):
    src = edge_index[0]
    dst = edge_index[1].reshape(NW, NCHUNK, K)
    zeros = jnp.zeros((N, D), jnp.float32)

    h = _bn_call(x, x_importance, bn_gamma.reshape(1, D), bn_beta.reshape(1, D))

    hs = []
    for W, b in ((W1, b1), (W2, b2), (W3, b3), (W4, b4)):
        agg = _agg_call(h, src, dst, zeros)
        h = _layer_call(h, agg, W, b.reshape(1, D))
        hs.append(h)

    agg = _agg_call(h, src, dst, zeros)
    h5, h6 = _last_call(h, agg, W5, b5.reshape(1, D), fc_w)

    return jnp.concatenate([hs[0], hs[1], hs[2], hs[3], h5, h6], axis=-1)


# speedup vs baseline: 8.1871x; 1.2806x over previous
import functools

import jax
import jax.numpy as jnp
from jax import lax
from jax.experimental import pallas as pl
from jax.experimental.pallas import tpu as pltpu
from jax.experimental.pallas import tpu_sc as plsc

N = 10000
E = 320000
D = 128

NC = 2
NS = 16
NW = NC * NS
EPW = E // NW
K = 80
NCHUNK = EPW // K


NBUF = 2


def _agg_body(h_hbm, src_hbm, dst_hbm, zero_hbm, out_hbm,
              src_v, dst_v, rows, acc, gsem, ssem):
    cid = lax.axis_index("c")
    sid = lax.axis_index("s")
    wid = sid * NC + cid

    pltpu.sync_copy(src_hbm.at[pl.ds(wid * EPW, EPW)], src_v)
    pltpu.sync_copy(dst_hbm.at[wid], dst_v)

    @pl.when(sid == 0)
    def _():
        pltpu.sync_copy(zero_hbm, acc)

    plsc.subcore_barrier()

    def gather_start(c, j):
        pltpu.async_copy(
            h_hbm.at[src_v.at[pl.ds(c * K, K)]], rows.at[j], gsem.at[j])

    def gather_wait(c, j):
        pltpu.make_async_copy(
            h_hbm.at[src_v.at[pl.ds(c * K, K)]], rows.at[j], gsem.at[j]).wait()

    def scatter_wait(c, j):
        pltpu.make_async_copy(rows.at[j], acc.at[dst_v.at[c]], ssem.at[j]).wait()

    gather_start(0, 0)

    def body(i, carry):
        for j in range(NBUF):
            c = i * NBUF + j
            jo = 1 - j
            gather_wait(c, j)
            pltpu.async_copy(rows.at[j], acc.at[dst_v.at[c]], ssem.at[j],
                             add=True)

            @pl.when(c >= 1)
            def _():
                scatter_wait(c - 1, jo)

            gather_start(c + 1, jo)
        return carry

    lax.fori_loop(0, (NCHUNK - 1) // NBUF, body, 0)

    gather_wait(NCHUNK - 1, 0)
    pltpu.async_copy(rows.at[0], acc.at[dst_v.at[NCHUNK - 1]], ssem.at[0],
                     add=True)
    scatter_wait(NCHUNK - 2, 1)
    scatter_wait(NCHUNK - 1, 0)

    plsc.subcore_barrier()

    @pl.when(sid == 0)
    def _():
        pltpu.sync_copy(acc, out_hbm.at[cid])


_agg_call = pl.kernel(
    _agg_body,
    out_type=jax.ShapeDtypeStruct((NC, N, D), jnp.float32),
    mesh=plsc.VectorSubcoreMesh(core_axis_name="c", subcore_axis_name="s"),
    scratch_types=[
        pltpu.VMEM((EPW,), jnp.int32),
        pltpu.VMEM((NCHUNK, K), jnp.int32),
        pltpu.VMEM((NBUF, K, D), jnp.float32),
        pltpu.VMEM_SHARED((N, D), jnp.float32),
        pltpu.SemaphoreType.DMA((NBUF,)),
        pltpu.SemaphoreType.DMA((NBUF,)),
    ],
)


def _bn_kernel(x_ref, imp_ref, g_ref, b_ref, o_ref):
    xs = x_ref[...] * imp_ref[...]
    mean = jnp.mean(xs, axis=0, keepdims=True)
    cent = xs - mean
    var = jnp.mean(cent * cent, axis=0, keepdims=True)
    o_ref[...] = cent * lax.rsqrt(var + 1e-5) * g_ref[...] + b_ref[...]


_bn_call = pl.pallas_call(
    _bn_kernel,
    out_shape=jax.ShapeDtypeStruct((N, D), jnp.float32),
)


BN_ROWS = 2000


def _layer_kernel(h_ref, a_ref, w_ref, b_ref, o_ref):
    s = h_ref[...] + a_ref[0] + a_ref[1]
    o_ref[...] = jnp.tanh(
        jnp.dot(s, w_ref[...], preferred_element_type=jnp.float32) + b_ref[...])


_layer_call = pl.pallas_call(
    _layer_kernel,
    grid=(N // BN_ROWS,),
    in_specs=[
        pl.BlockSpec((BN_ROWS, D), lambda i: (i, 0)),
        pl.BlockSpec((NC, BN_ROWS, D), lambda i: (0, i, 0)),
        pl.BlockSpec((D, D), lambda i: (0, 0)),
        pl.BlockSpec((1, D), lambda i: (0, 0)),
    ],
    out_specs=pl.BlockSpec((BN_ROWS, D), lambda i: (i, 0)),
    out_shape=jax.ShapeDtypeStruct((N, D), jnp.float32),
)


def _last_kernel(h_ref, a_ref, w_ref, b_ref, fc_ref, h5_ref, h6_ref):
    s = h_ref[...] + a_ref[0] + a_ref[1]
    h5 = jnp.tanh(
        jnp.dot(s, w_ref[...], preferred_element_type=jnp.float32) + b_ref[...])
    h5_ref[...] = h5
    h6_ref[...] = jnp.tanh(
        jnp.dot(h5, fc_ref[...], preferred_element_type=jnp.float32))


_last_call = pl.pallas_call(
    _last_kernel,
    grid=(N // BN_ROWS,),
    in_specs=[
        pl.BlockSpec((BN_ROWS, D), lambda i: (i, 0)),
        pl.BlockSpec((NC, BN_ROWS, D), lambda i: (0, i, 0)),
        pl.BlockSpec((D, D), lambda i: (0, 0)),
        pl.BlockSpec((1, D), lambda i: (0, 0)),
        pl.BlockSpec((D, D), lambda i: (0, 0)),
    ],
    out_specs=[
        pl.BlockSpec((BN_ROWS, D), lambda i: (i, 0)),
        pl.BlockSpec((BN_ROWS, D), lambda i: (i, 0)),
    ],
    out_shape=[
        jax.ShapeDtypeStruct((N, D), jnp.float32),
        jax.ShapeDtypeStruct((N, D), jnp.float32),
    ],
)


def kernel(x, x_importance, edge_index, bn_gamma, bn_beta,
           W1, b1, W2, b2, W3, b3, W4, b4, W5, b5, fc_w):
    src = edge_index[0]
    dst = edge_index[1].reshape(NW, NCHUNK, K)
    zeros = jnp.zeros((N, D), jnp.float32)

    h = _bn_call(x, x_importance, bn_gamma.reshape(1, D), bn_beta.reshape(1, D))

    hs = []
    for W, b in ((W1, b1), (W2, b2), (W3, b3), (W4, b4)):
        agg = _agg_call(h, src, dst, zeros)
        h = _layer_call(h, agg, W, b.reshape(1, D))
        hs.append(h)

    agg = _agg_call(h, src, dst, zeros)
    h5, h6 = _last_call(h, agg, W5, b5.reshape(1, D), fc_w)

    return jnp.concatenate([hs[0], hs[1], hs[2], hs[3], h5, h6], axis=-1)


# speedup vs baseline: 10.5868x; 1.2931x over previous
import functools

import jax
import jax.numpy as jnp
from jax import lax
from jax.experimental import pallas as pl
from jax.experimental.pallas import tpu as pltpu
from jax.experimental.pallas import tpu_sc as plsc

N = 10000
E = 320000
D = 128

NC = 2
NS = 16
NW = NC * NS
EPW = E // NW
K = 80
NCHUNK = EPW // K


NBUF = 4
NIDX = 8
RPT = 632
RPT_LAST = N - (NS - 1) * RPT


def _agg_body(h_hbm, eidx_hbm, zero_hbm, out_hbm,
              idx_v, rows, acc, isem, gsem, ssem):
    cid = lax.axis_index("c")
    sid = lax.axis_index("s")
    wid = sid * NC + cid

    def idx_start(c, ji):
        pltpu.async_copy(eidx_hbm.at[wid, c], idx_v.at[pl.ds(8 * ji, 2)],
                         isem.at[ji])

    def idx_wait(c, ji):
        pltpu.make_async_copy(eidx_hbm.at[wid, c], idx_v.at[pl.ds(8 * ji, 2)],
                              isem.at[ji]).wait()

    def gather_start(c, jr, ji):
        pltpu.async_copy(h_hbm.at[idx_v.at[8 * ji]], rows.at[jr], gsem.at[jr])

    def gather_wait(c, jr, ji):
        pltpu.make_async_copy(h_hbm.at[idx_v.at[8 * ji]], rows.at[jr],
                              gsem.at[jr]).wait()

    def scatter_start(c, jr, ji):
        pltpu.async_copy(rows.at[jr], acc.at[idx_v.at[8 * ji + 1]],
                         ssem.at[jr], add=True)

    def scatter_wait(c, jr, ji):
        pltpu.make_async_copy(rows.at[jr], acc.at[idx_v.at[8 * ji + 1]],
                              ssem.at[jr]).wait()

    for t in range(NBUF):
        idx_start(t, t)
    @pl.when(sid < NS - 1)
    def _():
        pltpu.sync_copy(zero_hbm.at[pl.ds(sid * RPT, RPT)],
                        acc.at[pl.ds(sid * RPT, RPT)])

    @pl.when(sid == NS - 1)
    def _():
        pltpu.sync_copy(zero_hbm.at[pl.ds((NS - 1) * RPT, RPT_LAST)],
                        acc.at[pl.ds((NS - 1) * RPT, RPT_LAST)])
    idx_wait(0, 0)
    gather_start(0, 0, 0)
    idx_wait(1, 1)
    gather_start(1, 1, 1)
    plsc.subcore_barrier()

    def step(c, j8):
        jr = j8 % NBUF
        gather_wait(c, jr, j8)
        scatter_start(c, jr, j8)

        @pl.when(c >= 2)
        def _():
            scatter_wait(c - 2, (j8 + 2) % NBUF, (j8 + 6) % NIDX)

        ji2 = (j8 + 2) % NIDX
        idx_wait(c + 2, ji2)
        gather_start(c + 2, (j8 + 2) % NBUF, ji2)
        idx_start(c + 4, (j8 + 4) % NIDX)

    def body(i, carry):
        for j8 in range(NIDX):
            step(i * NIDX + j8, j8)
        return carry

    NSTEady = (NCHUNK - 5) // NIDX * NIDX
    lax.fori_loop(0, NSTEady // NIDX, body, 0)

    for c in range(NSTEady, NCHUNK):
        j8 = c % NIDX
        jr = c % NBUF
        gather_wait(c, jr, j8)
        scatter_start(c, jr, j8)
        scatter_wait(c - 2, (c - 2) % NBUF, (c - 2) % NIDX)
        if c + 2 < NCHUNK:
            idx_wait(c + 2, (c + 2) % NIDX)
            gather_start(c + 2, (c + 2) % NBUF, (c + 2) % NIDX)
        if c + 4 < NCHUNK:
            idx_start(c + 4, (c + 4) % NIDX)
    scatter_wait(NCHUNK - 2, (NCHUNK - 2) % NBUF, (NCHUNK - 2) % NIDX)
    scatter_wait(NCHUNK - 1, (NCHUNK - 1) % NBUF, (NCHUNK - 1) % NIDX)

    plsc.subcore_barrier()

    @pl.when(sid < NS - 1)
    def _():
        pltpu.sync_copy(acc.at[pl.ds(sid * RPT, RPT)],
                        out_hbm.at[cid].at[pl.ds(sid * RPT, RPT)])

    @pl.when(sid == NS - 1)
    def _():
        pltpu.sync_copy(acc.at[pl.ds((NS - 1) * RPT, RPT_LAST)],
                        out_hbm.at[cid].at[pl.ds((NS - 1) * RPT, RPT_LAST)])


_agg_call = pl.kernel(
    _agg_body,
    out_type=jax.ShapeDtypeStruct((NC, N, D), jnp.float32),
    mesh=plsc.VectorSubcoreMesh(core_axis_name="c", subcore_axis_name="s"),
    scratch_types=[
        pltpu.VMEM((8 * NIDX, K), jnp.int32),
        pltpu.VMEM((NBUF, K, D), jnp.float32),
        pltpu.VMEM_SHARED((N, D), jnp.float32),
        pltpu.SemaphoreType.DMA((NIDX,)),
        pltpu.SemaphoreType.DMA((NBUF,)),
        pltpu.SemaphoreType.DMA((NBUF,)),
    ],
)


def _bn_kernel(x_ref, imp_ref, g_ref, b_ref, o_ref):
    xs = x_ref[...] * imp_ref[...]
    mean = jnp.mean(xs, axis=0, keepdims=True)
    cent = xs - mean
    var = jnp.mean(cent * cent, axis=0, keepdims=True)
    o_ref[...] = cent * lax.rsqrt(var + 1e-5) * g_ref[...] + b_ref[...]


_bn_call = pl.pallas_call(
    _bn_kernel,
    out_shape=jax.ShapeDtypeStruct((N, D), jnp.float32),
)


BN_ROWS = 2000


def _layer_kernel(h_ref, a_ref, w_ref, b_ref, o_ref):
    s = h_ref[...] + a_ref[0] + a_ref[1]
    o_ref[...] = jnp.tanh(
        jnp.dot(s, w_ref[...], preferred_element_type=jnp.float32) + b_ref[...])


_layer_call = pl.pallas_call(
    _layer_kernel,
    grid=(N // BN_ROWS,),
    in_specs=[
        pl.BlockSpec((BN_ROWS, D), lambda i: (i, 0)),
        pl.BlockSpec((NC, BN_ROWS, D), lambda i: (0, i, 0)),
        pl.BlockSpec((D, D), lambda i: (0, 0)),
        pl.BlockSpec((1, D), lambda i: (0, 0)),
    ],
    out_specs=pl.BlockSpec((BN_ROWS, D), lambda i: (i, 0)),
    out_shape=jax.ShapeDtypeStruct((N, D), jnp.float32),
)


def _last_kernel(h_ref, a_ref, w_ref, b_ref, fc_ref, h5_ref, h6_ref):
    s = h_ref[...] + a_ref[0] + a_ref[1]
    h5 = jnp.tanh(
        jnp.dot(s, w_ref[...], preferred_element_type=jnp.float32) + b_ref[...])
    h5_ref[...] = h5
    h6_ref[...] = jnp.tanh(
        jnp.dot(h5, fc_ref[...], preferred_element_type=jnp.float32))


_last_call = pl.pallas_call(
    _last_kernel,
    grid=(N // BN_ROWS,),
    in_specs=[
        pl.BlockSpec((BN_ROWS, D), lambda i: (i, 0)),
        pl.BlockSpec((NC, BN_ROWS, D), lambda i: (0, i, 0)),
        pl.BlockSpec((D, D), lambda i: (0, 0)),
        pl.BlockSpec((1, D), lambda i: (0, 0)),
        pl.BlockSpec((D, D), lambda i: (0, 0)),
    ],
    out_specs=[
        pl.BlockSpec((BN_ROWS, D), lambda i: (i, 0)),
        pl.BlockSpec((BN_ROWS, D), lambda i: (i, 0)),
    ],
    out_shape=[
        jax.ShapeDtypeStruct((N, D), jnp.float32),
        jax.ShapeDtypeStruct((N, D), jnp.float32),
    ],
)


def kernel(x, x_importance, edge_index, bn_gamma, bn_beta,
           W1, b1, W2, b2, W3, b3, W4, b4, W5, b5, fc_w):
    eidx = jnp.stack([edge_index[0].reshape(NW, NCHUNK, K),
                      edge_index[1].reshape(NW, NCHUNK, K)], axis=2)
    zeros = jnp.zeros((N, D), jnp.float32)

    h = _bn_call(x, x_importance, bn_gamma.reshape(1, D), bn_beta.reshape(1, D))

    hs = []
    for W, b in ((W1, b1), (W2, b2), (W3, b3), (W4, b4)):
        agg = _agg_call(h, eidx, zeros)
        h = _layer_call(h, agg, W, b.reshape(1, D))
        hs.append(h)

    agg = _agg_call(h, eidx, zeros)
    h5, h6 = _last_call(h, agg, W5, b5.reshape(1, D), fc_w)

    return jnp.concatenate([hs[0], hs[1], hs[2], hs[3], h5, h6], axis=-1)
